# Initial kernel scaffold; baseline (speedup 1.0000x reference)
import jax
import jax.numpy as jnp
from jax.experimental import pallas as pl


def kernel(x, edge_index, batch, W1_rel, W1_root, b1, W2_rel, W2_root, b2):
    raise NotImplementedError("write your pallas kernel here")


# speedup vs baseline: 25.4098x; 25.4098x over previous
import functools

import jax
import jax.numpy as jnp
from jax import lax
from jax.experimental import pallas as pl
from jax.experimental.pallas import tpu as pltpu
from jax.experimental.pallas import tpu_sc as plsc

NC = 2
NS = 16
NW = NC * NS
CH = 80


def _seg_sum_sc(feat, pk3, with_deg):
    n, d = feat.shape
    cpt = pk3.shape[1]
    rps = n // NS
    zr = 128
    mesh = plsc.VectorSubcoreMesh(core_axis_name="c", subcore_axis_name="s")

    out_type = [jax.ShapeDtypeStruct((NC, n, d), jnp.float32)]
    if with_deg:
        out_type.append(jax.ShapeDtypeStruct((NC, n), jnp.float32))

    @functools.partial(
        pl.kernel,
        out_type=out_type,
        mesh=mesh,
        scratch_types=[
            pltpu.VMEM((CH,), jnp.int32),
            pltpu.VMEM((CH,), jnp.int32),
            pltpu.VMEM((CH,), jnp.int32),
            pltpu.VMEM((CH, d), jnp.float32),
            pltpu.VMEM((zr, d), jnp.float32),
            pltpu.VMEM((CH,), jnp.float32),
            pltpu.VMEM((rps,), jnp.float32),
            pltpu.VMEM_SHARED((n, d), jnp.float32),
            pltpu.VMEM_SHARED((n,), jnp.float32),
            pltpu.SemaphoreType.DMA,
        ],
    )
    def k(feat_hbm, pk_hbm, acc_hbm, *rest):
        if with_deg:
            (deg_hbm, pbuf, sidx, didx, gbuf, zbuf, ones_v, dbuf, acc, deg,
             sem) = rest
        else:
            (pbuf, sidx, didx, gbuf, zbuf, ones_v, dbuf, acc, deg,
             sem) = rest
        c = lax.axis_index("c")
        s = lax.axis_index("s")
        wid = s * NC + c

        zeros16 = jnp.zeros((16,), jnp.float32)
        ones16 = jnp.ones((16,), jnp.float32)

        def zero_body(i, _):
            for j in range(d // 16):
                zbuf[i, pl.ds(j * 16, 16)] = zeros16
            return 0

        lax.fori_loop(0, zr, zero_body, 0)

        def zero1d_body(i, _):
            dbuf[pl.ds(i * 16, 16)] = zeros16
            return 0

        lax.fori_loop(0, rps // 16, zero1d_body, 0)
        for j in range(CH // 16):
            ones_v[pl.ds(j * 16, 16)] = ones16

        for kk in range(rps // zr):
            pltpu.sync_copy(zbuf, acc.at[pl.ds(s * rps + kk * zr, zr), :])
        if with_deg:
            pltpu.sync_copy(dbuf, deg.at[pl.ds(s * rps, rps)])
        plsc.subcore_barrier()

        def edge_body(i, _):
            pltpu.sync_copy(pk_hbm.at[wid, i], pbuf)
            for j in range(CH // 16):
                v = pbuf[pl.ds(j * 16, 16)]
                sidx[pl.ds(j * 16, 16)] = lax.bitwise_and(v, 0xFFFF)
                didx[pl.ds(j * 16, 16)] = lax.shift_right_logical(v, 16)
            pltpu.async_copy(feat_hbm.at[sidx], gbuf, sem).wait()
            pltpu.sync_copy(gbuf, acc.at[didx], add=True)
            if with_deg:
                pltpu.sync_copy(ones_v, deg.at[didx], add=True)
            return 0

        lax.fori_loop(0, cpt, edge_body, 0)
        plsc.subcore_barrier()

        for kk in range(rps // zr):
            base = s * rps + kk * zr
            pltpu.sync_copy(acc.at[pl.ds(base, zr), :], zbuf)
            pltpu.sync_copy(zbuf, acc_hbm.at[c, pl.ds(base, zr), :])
        if with_deg:
            pltpu.sync_copy(deg.at[pl.ds(s * rps, rps)], dbuf)
            pltpu.sync_copy(dbuf, deg_hbm.at[c, pl.ds(s * rps, rps)])

    return k(feat, pk3)


def _tc_layer1(x, p0, p1, d0, d1, w1_root, b1, w1_rel0, w2_rel0, w2_root, b2):
    n, din = x.shape
    hid = w1_root.shape[1]
    dout = w2_root.shape[1]
    blk = 1024
    grid = (n // blk,)

    def body(x_ref, p0_ref, p1_ref, d0_ref, d1_ref, w1r_ref, b1_ref, w1e_ref,
             w2e_ref, w2r_ref, b2_ref, hr_ref, hroot_ref, invb_ref):
        agg = p0_ref[...] + p1_ref[...]
        deg = d0_ref[...] + d1_ref[...]
        inv = 1.0 / jnp.maximum(deg, 1.0)
        h = x_ref[...] @ w1r_ref[...] + (agg * inv) @ w1e_ref[...] + b1_ref[...]
        h = jnp.maximum(h, 0.0)
        hr_ref[...] = h @ w2e_ref[...]
        hroot_ref[...] = h @ w2r_ref[...] + b2_ref[...]
        invb_ref[...] = jnp.broadcast_to(inv, (blk, dout))

    return pl.pallas_call(
        body,
        grid=grid,
        in_specs=[
            pl.BlockSpec((blk, din), lambda i: (i, 0)),
            pl.BlockSpec((blk, din), lambda i: (i, 0)),
            pl.BlockSpec((blk, din), lambda i: (i, 0)),
            pl.BlockSpec((blk, 1), lambda i: (i, 0)),
            pl.BlockSpec((blk, 1), lambda i: (i, 0)),
            pl.BlockSpec((din, hid), lambda i: (0, 0)),
            pl.BlockSpec((1, hid), lambda i: (0, 0)),
            pl.BlockSpec((din, hid), lambda i: (0, 0)),
            pl.BlockSpec((hid, dout), lambda i: (0, 0)),
            pl.BlockSpec((hid, dout), lambda i: (0, 0)),
            pl.BlockSpec((1, dout), lambda i: (0, 0)),
        ],
        out_specs=[
            pl.BlockSpec((blk, dout), lambda i: (i, 0)),
            pl.BlockSpec((blk, dout), lambda i: (i, 0)),
            pl.BlockSpec((blk, dout), lambda i: (i, 0)),
        ],
        out_shape=[
            jax.ShapeDtypeStruct((n, dout), jnp.float32),
            jax.ShapeDtypeStruct((n, dout), jnp.float32),
            jax.ShapeDtypeStruct((n, dout), jnp.float32),
        ],
    )(x, p0, p1, d0, d1, w1_root, b1.reshape(1, hid), w1_rel0, w2_rel0,
      w2_root, b2.reshape(1, dout))


def _tc_pool(hroot, p0, p1, invb, batch3, n_graphs):
    n, d = hroot.shape
    blk = 1024
    grid = (n // blk,)

    def body(hroot_ref, p0_ref, p1_ref, invb_ref, b_ref, out_ref, acc_sum,
             acc_cnt):
        i = pl.program_id(0)
        node = hroot_ref[...] + (p0_ref[...] + p1_ref[...]) * invb_ref[...]
        bid = b_ref[...].reshape(1, blk)
        gids = lax.broadcasted_iota(jnp.int32, (n_graphs, blk), 0)
        mask = (jnp.broadcast_to(bid, (n_graphs, blk)) == gids).astype(
            jnp.float32)
        psum = mask @ node
        pcnt = mask @ jnp.ones((blk, d), jnp.float32)

        @pl.when(i == 0)
        def _():
            acc_sum[...] = psum
            acc_cnt[...] = pcnt

        @pl.when(i > 0)
        def _():
            acc_sum[...] += psum
            acc_cnt[...] += pcnt

        @pl.when(i == grid[0] - 1)
        def _():
            out_ref[...] = acc_sum[...] / jnp.maximum(acc_cnt[...], 1.0)

    return pl.pallas_call(
        body,
        grid=grid,
        in_specs=[
            pl.BlockSpec((blk, d), lambda i: (i, 0)),
            pl.BlockSpec((blk, d), lambda i: (i, 0)),
            pl.BlockSpec((blk, d), lambda i: (i, 0)),
            pl.BlockSpec((blk, d), lambda i: (i, 0)),
            pl.BlockSpec((1, 1, blk), lambda i: (i, 0, 0)),
        ],
        out_specs=pl.BlockSpec((n_graphs, d), lambda i: (0, 0)),
        out_shape=jax.ShapeDtypeStruct((n_graphs, d), jnp.float32),
        scratch_shapes=[
            pltpu.VMEM((n_graphs, d), jnp.float32),
            pltpu.VMEM((n_graphs, d), jnp.float32),
        ],
    )(hroot, p0, p1, invb, batch3)


def kernel(x, edge_index, batch, W1_rel, W1_root, b1, W2_rel, W2_root, b2):
    n, din = x.shape
    e = edge_index.shape[1]
    n_graphs = 16
    cpt = e // (NW * CH)

    np_ = ((n + 1023) // 1024) * 1024
    x_p = jnp.pad(x, ((0, np_ - n), (0, 0)))
    batch_p = jnp.pad(batch, (0, np_ - n), constant_values=n_graphs)

    packed = jnp.bitwise_or(edge_index[0],
                            jnp.left_shift(edge_index[1], 16))
    pk3 = packed.reshape(NW, cpt, CH)

    parts1, degs1 = _seg_sum_sc(x_p, pk3, True)
    d0 = degs1[0].reshape(np_, 1)
    d1 = degs1[1].reshape(np_, 1)
    hr, hroot, invb = _tc_layer1(x_p, parts1[0], parts1[1], d0, d1, W1_root,
                                 b1, W1_rel[0], W2_rel[0], W2_root, b2)
    (parts2,) = _seg_sum_sc(hr, pk3, False)
    batch3 = batch_p.reshape(np_ // 1024, 1, 1024)
    return _tc_pool(hroot, parts2[0], parts2[1], invb, batch3, n_graphs)


# speedup vs baseline: 49.2596x; 1.9386x over previous
import functools

import jax
import jax.numpy as jnp
from jax import lax
from jax.experimental import pallas as pl
from jax.experimental.pallas import tpu as pltpu
from jax.experimental.pallas import tpu_sc as plsc

NC = 2
NS = 16
NW = NC * NS
CH = 80


def _seg_sum_sc(feat, pk3, with_deg):
    n, d = feat.shape
    cpt = pk3.shape[1]
    rps = n // NS
    zr = 128
    mesh = plsc.VectorSubcoreMesh(core_axis_name="c", subcore_axis_name="s")

    out_type = [jax.ShapeDtypeStruct((NC, n, d), jnp.float32)]
    if with_deg:
        out_type.append(jax.ShapeDtypeStruct((NC, n), jnp.float32))

    @functools.partial(
        pl.kernel,
        out_type=out_type,
        mesh=mesh,
        scratch_types=[
            pltpu.VMEM((2, CH), jnp.int32),
            pltpu.VMEM((2, CH), jnp.int32),
            pltpu.VMEM((2, CH), jnp.int32),
            pltpu.VMEM((2, CH, d), jnp.float32),
            pltpu.VMEM((zr, d), jnp.float32),
            pltpu.VMEM((CH,), jnp.float32),
            pltpu.VMEM((rps,), jnp.float32),
            pltpu.VMEM_SHARED((n, d), jnp.float32),
            pltpu.VMEM_SHARED((n,), jnp.float32),
        ] + [pltpu.SemaphoreType.DMA] * 8,
    )
    def k(feat_hbm, pk_hbm, acc_hbm, *rest):
        if with_deg:
            (deg_hbm, pbuf, sidx, didx, gbuf, zbuf, ones_v, dbuf, acc, deg,
             *sems) = rest
        else:
            (pbuf, sidx, didx, gbuf, zbuf, ones_v, dbuf, acc, deg,
             *sems) = rest
        sem_i = sems[0:2]
        sem_g = sems[2:4]
        sem_s = sems[4:6]
        sem_d = sems[6:8]
        c = lax.axis_index("c")
        s = lax.axis_index("s")
        wid = s * NC + c

        zeros16 = jnp.zeros((16,), jnp.float32)
        ones16 = jnp.ones((16,), jnp.float32)

        def zero_body(i, _):
            for j in range(d // 16):
                zbuf[i, pl.ds(j * 16, 16)] = zeros16
            return 0

        lax.fori_loop(0, zr, zero_body, 0)

        def zero1d_body(i, _):
            dbuf[pl.ds(i * 16, 16)] = zeros16
            return 0

        lax.fori_loop(0, rps // 16, zero1d_body, 0)
        for j in range(CH // 16):
            ones_v[pl.ds(j * 16, 16)] = ones16

        for kk in range(rps // zr):
            pltpu.sync_copy(zbuf, acc.at[pl.ds(s * rps + kk * zr, zr), :])
        if with_deg:
            pltpu.sync_copy(dbuf, deg.at[pl.ds(s * rps, rps)])
        plsc.subcore_barrier()

        def load(cc, b):
            pltpu.async_copy(pk_hbm.at[wid, cc], pbuf.at[b], sem_i[b])

        def prep(cc, b):
            @pl.when(cc >= 2)
            def _():
                pltpu.make_async_copy(gbuf.at[b], acc.at[didx.at[b]],
                                      sem_s[b]).wait()
                if with_deg:
                    pltpu.make_async_copy(ones_v, deg.at[didx.at[b]],
                                          sem_d[b]).wait()
            pltpu.make_async_copy(pk_hbm.at[wid, cc], pbuf.at[b],
                                  sem_i[b]).wait()
            for j in range(CH // 16):
                v = pbuf[b, pl.ds(j * 16, 16)]
                sidx[b, pl.ds(j * 16, 16)] = lax.bitwise_and(v, 0xFFFF)
                didx[b, pl.ds(j * 16, 16)] = lax.shift_right_logical(v, 16)
            pltpu.async_copy(feat_hbm.at[sidx.at[b]], gbuf.at[b], sem_g[b])

        def fin(cc, b):
            pltpu.make_async_copy(feat_hbm.at[sidx.at[b]], gbuf.at[b],
                                  sem_g[b]).wait()
            pltpu.async_copy(gbuf.at[b], acc.at[didx.at[b]], sem_s[b],
                             add=True)
            if with_deg:
                pltpu.async_copy(ones_v, deg.at[didx.at[b]], sem_d[b],
                                 add=True)

        load(0, 0)
        load(1, 1)
        prep(0, 0)

        def pipe_body(g, _):
            for b in (0, 1):
                cc = 2 * g + b

                @pl.when(cc + 1 < cpt)
                def _():
                    prep(cc + 1, (b + 1) % 2)

                @pl.when(cc + 2 < cpt)
                def _():
                    load(cc + 2, b)

                @pl.when(cc < cpt)
                def _():
                    fin(cc, b)
            return 0

        lax.fori_loop(0, (cpt + 2) // 2, pipe_body, 0)

        for b in (0, 1):
            pltpu.make_async_copy(gbuf.at[b], acc.at[didx.at[b]],
                                  sem_s[b]).wait()
            if with_deg:
                pltpu.make_async_copy(ones_v, deg.at[didx.at[b]],
                                      sem_d[b]).wait()
        plsc.subcore_barrier()

        for kk in range(rps // zr):
            base = s * rps + kk * zr
            pltpu.sync_copy(acc.at[pl.ds(base, zr), :], zbuf)
            pltpu.sync_copy(zbuf, acc_hbm.at[c, pl.ds(base, zr), :])
        if with_deg:
            pltpu.sync_copy(deg.at[pl.ds(s * rps, rps)], dbuf)
            pltpu.sync_copy(dbuf, deg_hbm.at[c, pl.ds(s * rps, rps)])

    return k(feat, pk3)


def _tc_layer1(x, p0, p1, d0, d1, w1_root, b1, w1_rel0, w2_rel0, w2_root, b2):
    n, din = x.shape
    hid = w1_root.shape[1]
    dout = w2_root.shape[1]
    blk = 1024
    grid = (n // blk,)

    def body(x_ref, p0_ref, p1_ref, d0_ref, d1_ref, w1r_ref, b1_ref, w1e_ref,
             w2e_ref, w2r_ref, b2_ref, hr_ref, hroot_ref, invb_ref):
        agg = p0_ref[...] + p1_ref[...]
        deg = d0_ref[...] + d1_ref[...]
        inv = 1.0 / jnp.maximum(deg, 1.0)
        h = x_ref[...] @ w1r_ref[...] + (agg * inv) @ w1e_ref[...] + b1_ref[...]
        h = jnp.maximum(h, 0.0)
        hr_ref[...] = h @ w2e_ref[...]
        hroot_ref[...] = h @ w2r_ref[...] + b2_ref[...]
        invb_ref[...] = jnp.broadcast_to(inv, (blk, dout))

    return pl.pallas_call(
        body,
        grid=grid,
        in_specs=[
            pl.BlockSpec((blk, din), lambda i: (i, 0)),
            pl.BlockSpec((blk, din), lambda i: (i, 0)),
            pl.BlockSpec((blk, din), lambda i: (i, 0)),
            pl.BlockSpec((blk, 1), lambda i: (i, 0)),
            pl.BlockSpec((blk, 1), lambda i: (i, 0)),
            pl.BlockSpec((din, hid), lambda i: (0, 0)),
            pl.BlockSpec((1, hid), lambda i: (0, 0)),
            pl.BlockSpec((din, hid), lambda i: (0, 0)),
            pl.BlockSpec((hid, dout), lambda i: (0, 0)),
            pl.BlockSpec((hid, dout), lambda i: (0, 0)),
            pl.BlockSpec((1, dout), lambda i: (0, 0)),
        ],
        out_specs=[
            pl.BlockSpec((blk, dout), lambda i: (i, 0)),
            pl.BlockSpec((blk, dout), lambda i: (i, 0)),
            pl.BlockSpec((blk, dout), lambda i: (i, 0)),
        ],
        out_shape=[
            jax.ShapeDtypeStruct((n, dout), jnp.float32),
            jax.ShapeDtypeStruct((n, dout), jnp.float32),
            jax.ShapeDtypeStruct((n, dout), jnp.float32),
        ],
    )(x, p0, p1, d0, d1, w1_root, b1.reshape(1, hid), w1_rel0, w2_rel0,
      w2_root, b2.reshape(1, dout))


def _tc_pool(hroot, p0, p1, invb, batch3, n_graphs):
    n, d = hroot.shape
    blk = 1024
    grid = (n // blk,)

    def body(hroot_ref, p0_ref, p1_ref, invb_ref, b_ref, out_ref, acc_sum,
             acc_cnt):
        i = pl.program_id(0)
        node = hroot_ref[...] + (p0_ref[...] + p1_ref[...]) * invb_ref[...]
        bid = b_ref[...].reshape(1, blk)
        gids = lax.broadcasted_iota(jnp.int32, (n_graphs, blk), 0)
        mask = (jnp.broadcast_to(bid, (n_graphs, blk)) == gids).astype(
            jnp.float32)
        psum = mask @ node
        pcnt = mask @ jnp.ones((blk, d), jnp.float32)

        @pl.when(i == 0)
        def _():
            acc_sum[...] = psum
            acc_cnt[...] = pcnt

        @pl.when(i > 0)
        def _():
            acc_sum[...] += psum
            acc_cnt[...] += pcnt

        @pl.when(i == grid[0] - 1)
        def _():
            out_ref[...] = acc_sum[...] / jnp.maximum(acc_cnt[...], 1.0)

    return pl.pallas_call(
        body,
        grid=grid,
        in_specs=[
            pl.BlockSpec((blk, d), lambda i: (i, 0)),
            pl.BlockSpec((blk, d), lambda i: (i, 0)),
            pl.BlockSpec((blk, d), lambda i: (i, 0)),
            pl.BlockSpec((blk, d), lambda i: (i, 0)),
            pl.BlockSpec((1, 1, blk), lambda i: (i, 0, 0)),
        ],
        out_specs=pl.BlockSpec((n_graphs, d), lambda i: (0, 0)),
        out_shape=jax.ShapeDtypeStruct((n_graphs, d), jnp.float32),
        scratch_shapes=[
            pltpu.VMEM((n_graphs, d), jnp.float32),
            pltpu.VMEM((n_graphs, d), jnp.float32),
        ],
    )(hroot, p0, p1, invb, batch3)


def kernel(x, edge_index, batch, W1_rel, W1_root, b1, W2_rel, W2_root, b2):
    n, din = x.shape
    e = edge_index.shape[1]
    n_graphs = 16
    cpt = e // (NW * CH)

    np_ = ((n + 1023) // 1024) * 1024
    x_p = jnp.pad(x, ((0, np_ - n), (0, 0)))
    batch_p = jnp.pad(batch, (0, np_ - n), constant_values=n_graphs)

    packed = jnp.bitwise_or(edge_index[0],
                            jnp.left_shift(edge_index[1], 16))
    pk3 = packed.reshape(NW, cpt, CH)

    parts1, degs1 = _seg_sum_sc(x_p, pk3, True)
    d0 = degs1[0].reshape(np_, 1)
    d1 = degs1[1].reshape(np_, 1)
    hr, hroot, invb = _tc_layer1(x_p, parts1[0], parts1[1], d0, d1, W1_root,
                                 b1, W1_rel[0], W2_rel[0], W2_root, b2)
    (parts2,) = _seg_sum_sc(hr, pk3, False)
    batch3 = batch_p.reshape(np_ // 1024, 1, 1024)
    return _tc_pool(hroot, parts2[0], parts2[1], invb, batch3, n_graphs)
